# baseline (speedup 1.0000x reference)
import functools

import jax
import jax.numpy as jnp
from jax.experimental import pallas as pl
from jax.experimental.pallas import tpu as pltpu

TEMP = 0.05
BATCH = 1024
NUM_FEATURES = 64
NUM_SAMPLES = 100000
C_TILE = 2048
NUM_TILES = (NUM_SAMPLES + C_TILE - 1) // C_TILE

NEG_BIG = -1e30


def _ce_kernel(inputs_ref, targets_ref, feat_ref, out_ref, loss_ref,
               xn_ref, m_ref, s_ref, p_ref):
    i = pl.program_id(0)

    @pl.when(i == 0)
    def _init():
        x = inputs_ref[...]
        norm = jnp.sqrt(jnp.sum(x * x, axis=1, keepdims=True))
        xn_ref[...] = x / jnp.maximum(norm, 1e-12)
        m_ref[...] = jnp.full((BATCH, 1), NEG_BIG, jnp.float32)
        s_ref[...] = jnp.zeros((BATCH, 1), jnp.float32)
        p_ref[...] = jnp.zeros((BATCH, 1), jnp.float32)

    xn = xn_ref[...]
    logits = jax.lax.dot_general(
        xn, feat_ref[...],
        dimension_numbers=(((1,), (1,)), ((), ())),
        preferred_element_type=jnp.float32,
    ) * jnp.float32(1.0 / TEMP)

    col0 = i * C_TILE
    cols = col0 + jax.lax.broadcasted_iota(jnp.int32, (BATCH, C_TILE), 1)
    valid = cols < NUM_SAMPLES
    logits = jnp.where(valid, logits, NEG_BIG)
    out_ref[...] = logits

    tile_max = jnp.max(logits, axis=1, keepdims=True)
    m_old = m_ref[...]
    m_new = jnp.maximum(m_old, tile_max)
    s_ref[...] = (s_ref[...] * jnp.exp(m_old - m_new)
                  + jnp.sum(jnp.exp(logits - m_new), axis=1, keepdims=True))
    m_ref[...] = m_new

    tgt = targets_ref[...]
    hit = cols == tgt
    p_ref[...] += jnp.sum(jnp.where(hit, logits, 0.0), axis=1, keepdims=True)

    @pl.when(i == NUM_TILES - 1)
    def _fin():
        lse = m_ref[...] + jnp.log(s_ref[...])
        loss = -jnp.mean(p_ref[...] - lse)
        loss = jnp.where(jnp.isnan(loss), jnp.float32(0.0), loss)
        loss_ref[...] = jnp.reshape(loss, (1, 1))


@jax.jit
def _run(inputs, targets, features):
    out, loss = pl.pallas_call(
        _ce_kernel,
        grid=(NUM_TILES,),
        in_specs=[
            pl.BlockSpec((BATCH, NUM_FEATURES), lambda i: (0, 0)),
            pl.BlockSpec((BATCH, 1), lambda i: (0, 0)),
            pl.BlockSpec((C_TILE, NUM_FEATURES), lambda i: (i, 0)),
        ],
        out_specs=[
            pl.BlockSpec((BATCH, C_TILE), lambda i: (0, i)),
            pl.BlockSpec((1, 1), lambda i: (0, 0)),
        ],
        out_shape=[
            jax.ShapeDtypeStruct((BATCH, NUM_SAMPLES), jnp.float32),
            jax.ShapeDtypeStruct((1, 1), jnp.float32),
        ],
        scratch_shapes=[
            pltpu.VMEM((BATCH, NUM_FEATURES), jnp.float32),
            pltpu.VMEM((BATCH, 1), jnp.float32),
            pltpu.VMEM((BATCH, 1), jnp.float32),
            pltpu.VMEM((BATCH, 1), jnp.float32),
        ],
    )(inputs, targets.astype(jnp.int32).reshape(BATCH, 1), features)
    return loss[0, 0], out


def kernel(inputs, targets, features):
    loss, out = _run(inputs, targets, features)
    return (loss, out)
